# Initial kernel scaffold; baseline (speedup 1.0000x reference)
import jax
import jax.numpy as jnp
from jax.experimental import pallas as pl


def kernel(x_nchw, w1, b1, w2, b2):
    raise NotImplementedError("write your pallas kernel here")


# speedup vs baseline: 1.3942x; 1.3942x over previous
import functools

import jax
import jax.numpy as jnp
from jax import lax
from jax.experimental import pallas as pl
from jax.experimental.pallas import tpu as pltpu


def _resblock_kernel(x_ref, w_ref, b_ref, o_ref, *, W, HW):
    NB, C, _ = x_ref.shape
    L = NB * HW

    x = jnp.concatenate([x_ref[i] for i in range(NB)], axis=1)

    col = lax.broadcasted_iota(jnp.int32, (C, L), 1)
    w_pos = col & (W - 1)
    hw_pos = col & (HW - 1)
    is_w_first = w_pos == 0
    is_w_last = w_pos == W - 1
    is_h_first = hw_pos < W
    is_h_last = hw_pos >= HW - W

    def lane_roll(a, shift):
        return pltpu.roll(a, shift % L, axis=1)

    def tap_slab(src):
        left = lane_roll(src, 1)
        right = lane_roll(src, -1)
        w_taps = (jnp.where(is_w_first, right, left), src,
                  jnp.where(is_w_last, left, right))
        cols = []
        for kw in range(3):
            a = w_taps[kw]
            up = lane_roll(a, W)
            down = lane_roll(a, -W)
            cols.append((jnp.where(is_h_first, down, up), a,
                         jnp.where(is_h_last, up, down)))
        return jnp.concatenate(
            [cols[kw][kh] for kh in range(3) for kw in range(3)], axis=0)

    def conv3x3(src_bf16, j):
        return jnp.dot(w_ref[j], tap_slab(src_bf16),
                       preferred_element_type=jnp.float32) + b_ref[j]

    h1 = jnp.maximum(conv3x3(x.astype(jnp.bfloat16), 0), 0.0)
    out = jnp.maximum(conv3x3(h1.astype(jnp.bfloat16), 1) + x, 0.0)

    for i in range(NB):
        o_ref[i] = out[:, i * HW:(i + 1) * HW].astype(o_ref.dtype)


def kernel(x_nchw, w1, b1, w2, b2):
    N, C, H, W = x_nchw.shape
    HW = H * W

    nb = next(d for d in (8, 4, 2, 1) if N % d == 0)
    ng = N // nb

    x_packed = x_nchw.reshape(N, C, HW)

    def pack_w(w):
        taps = jnp.transpose(w, (2, 3, 0, 1)).reshape(9, C, C)
        return jnp.transpose(taps, (1, 0, 2)).reshape(C, 9 * C)

    w_all = jnp.stack([pack_w(w1), pack_w(w2)]).astype(jnp.bfloat16)
    b_all = jnp.stack([b1.reshape(C, 1), b2.reshape(C, 1)]).astype(jnp.float32)

    fn = functools.partial(_resblock_kernel, W=W, HW=HW)

    out = pl.pallas_call(
        fn,
        out_shape=jax.ShapeDtypeStruct((N, C, HW), x_nchw.dtype),
        grid=(ng,),
        in_specs=[
            pl.BlockSpec((nb, C, HW), lambda g: (g, 0, 0)),
            pl.BlockSpec((2, C, 9 * C), lambda g: (0, 0, 0)),
            pl.BlockSpec((2, C, 1), lambda g: (0, 0, 0)),
        ],
        out_specs=pl.BlockSpec((nb, C, HW), lambda g: (g, 0, 0)),
        compiler_params=pltpu.CompilerParams(
            dimension_semantics=("parallel",)),
    )(x_packed, w_all, b_all)

    return out.reshape(N, C, H, W)


# speedup vs baseline: 1.4531x; 1.0422x over previous
import functools

import jax
import jax.numpy as jnp
from jax import lax
from jax.experimental import pallas as pl
from jax.experimental.pallas import tpu as pltpu


def _resblock_kernel(x_ref, w_ref, b_ref, o_ref, *, W, HW, GRP):
    NB, C, _ = x_ref.shape
    L = GRP * HW

    col = lax.broadcasted_iota(jnp.int16, (C, L), 1)
    w_pos = col & jnp.int16(W - 1)
    hw_pos = col & jnp.int16(HW - 1)
    is_w_first = w_pos == 0
    is_w_last = w_pos == W - 1
    is_h_first = hw_pos < W
    is_h_last = hw_pos >= HW - W

    def lane_roll(a, shift):
        return pltpu.roll(a, shift % L, axis=1)

    def tap_slab(src):
        left = lane_roll(src, 1)
        right = lane_roll(src, -1)
        w_taps = (jnp.where(is_w_first, right, left), src,
                  jnp.where(is_w_last, left, right))
        cols = []
        for kw in range(3):
            a = w_taps[kw]
            up = lane_roll(a, W)
            down = lane_roll(a, -W)
            cols.append((jnp.where(is_h_first, down, up), a,
                         jnp.where(is_h_last, up, down)))
        return jnp.concatenate(
            [cols[kw][kh] for kh in range(3) for kw in range(3)], axis=0)

    def conv3x3(src_bf16, j):
        return jnp.dot(w_ref[j], tap_slab(src_bf16),
                       preferred_element_type=jnp.float32) + b_ref[j]

    def load(g0):
        return jnp.concatenate([x_ref[g0 + i] for i in range(GRP)], axis=1)

    def store(g0, out):
        for i in range(GRP):
            o_ref[g0 + i] = out[:, i * HW:(i + 1) * HW].astype(o_ref.dtype)

    nch = NB // GRP
    xs = [None] * nch
    d1 = [None] * nch
    d2 = [None] * nch

    def stage1(k):
        xs[k] = load(k * GRP)
        d1[k] = conv3x3(xs[k].astype(jnp.bfloat16), 0)

    def stage2(k):
        h1 = jnp.maximum(d1[k], 0.0)
        d2[k] = conv3x3(h1.astype(jnp.bfloat16), 1)

    def stage3(k):
        store(k * GRP, jnp.maximum(d2[k] + xs[k], 0.0))

    for t in range(nch + 2):
        if t < nch:
            stage1(t)
        if 1 <= t <= nch:
            stage2(t - 1)
        if t >= 2:
            stage3(t - 2)


def kernel(x_nchw, w1, b1, w2, b2):
    N, C, H, W = x_nchw.shape
    HW = H * W

    nb = next(d for d in (16, 8, 4, 2, 1) if N % d == 0)
    grp = max(nb // 2, 1)
    ng = N // nb

    x_packed = x_nchw.reshape(N, C, HW)

    def pack_w(w):
        taps = jnp.transpose(w, (2, 3, 0, 1)).reshape(9, C, C)
        return jnp.transpose(taps, (1, 0, 2)).reshape(C, 9 * C)

    w_all = jnp.stack([pack_w(w1), pack_w(w2)]).astype(jnp.bfloat16)
    b_all = jnp.stack([b1.reshape(C, 1), b2.reshape(C, 1)]).astype(jnp.float32)

    fn = functools.partial(_resblock_kernel, W=W, HW=HW, GRP=grp)

    out = pl.pallas_call(
        fn,
        out_shape=jax.ShapeDtypeStruct((N, C, HW), x_nchw.dtype),
        grid=(ng,),
        in_specs=[
            pl.BlockSpec((nb, C, HW), lambda g: (g, 0, 0)),
            pl.BlockSpec((2, C, 9 * C), lambda g: (0, 0, 0)),
            pl.BlockSpec((2, C, 1), lambda g: (0, 0, 0)),
        ],
        out_specs=pl.BlockSpec((nb, C, HW), lambda g: (g, 0, 0)),
        compiler_params=pltpu.CompilerParams(
            dimension_semantics=("parallel",)),
    )(x_packed, w_all, b_all)

    return out.reshape(N, C, H, W)


# speedup vs baseline: 1.5007x; 1.0328x over previous
import functools

import jax
import jax.numpy as jnp
from jax import lax
from jax.experimental import pallas as pl
from jax.experimental.pallas import tpu as pltpu


def _resblock_kernel(x_ref, w_ref, b_ref, o_ref, *, W, HW, GRP):
    NB, C, _ = x_ref.shape
    L = GRP * HW

    col = lax.broadcasted_iota(jnp.int16, (C, L), 1)
    w_pos = col & jnp.int16(W - 1)
    hw_pos = col & jnp.int16(HW - 1)
    is_w_first = w_pos == 0
    is_w_last = w_pos == W - 1
    is_h_first = hw_pos < W
    is_h_last = hw_pos >= HW - W

    def lane_roll(a, shift):
        return pltpu.roll(a, shift % L, axis=1)

    narrow = HW % 128 == 0 and HW >= 256
    if narrow:
        mh_first = is_h_first[:, :128]
        mh_last = is_h_last[:, HW - 128:HW]

    def h_first_fix(up, down):
        if not narrow:
            return jnp.where(is_h_first, down, up)
        pieces = []
        for i in range(0, L, HW):
            pieces.append(jnp.where(mh_first, down[:, i:i + 128],
                                    up[:, i:i + 128]))
            pieces.append(up[:, i + 128:i + HW])
        return jnp.concatenate(pieces, axis=1)

    def h_last_fix(up, down):
        if not narrow:
            return jnp.where(is_h_last, up, down)
        pieces = []
        for i in range(0, L, HW):
            pieces.append(down[:, i:i + HW - 128])
            pieces.append(jnp.where(mh_last, up[:, i + HW - 128:i + HW],
                                    down[:, i + HW - 128:i + HW]))
        return jnp.concatenate(pieces, axis=1)

    def tap_slab(src):
        left = lane_roll(src, 1)
        right = lane_roll(src, -1)
        w_taps = (jnp.where(is_w_first, right, left), src,
                  jnp.where(is_w_last, left, right))
        cols = []
        for kw in range(3):
            a = w_taps[kw]
            up = lane_roll(a, W)
            down = lane_roll(a, -W)
            cols.append((h_first_fix(up, down), a, h_last_fix(up, down)))
        return jnp.concatenate(
            [cols[kw][kh] for kh in range(3) for kw in range(3)], axis=0)

    def conv3x3(src_bf16, j):
        return jnp.dot(w_ref[j], tap_slab(src_bf16),
                       preferred_element_type=jnp.float32) + b_ref[j]

    def load(g0):
        return jnp.concatenate([x_ref[g0 + i] for i in range(GRP)], axis=1)

    def store(g0, out):
        for i in range(GRP):
            o_ref[g0 + i] = out[:, i * HW:(i + 1) * HW].astype(o_ref.dtype)

    nch = NB // GRP
    xs = [None] * nch
    d1 = [None] * nch
    d2 = [None] * nch

    def stage1(k):
        xs[k] = load(k * GRP)
        d1[k] = conv3x3(xs[k].astype(jnp.bfloat16), 0)

    def stage2(k):
        h1 = jnp.maximum(d1[k], 0.0)
        d2[k] = conv3x3(h1.astype(jnp.bfloat16), 1)

    def stage3(k):
        store(k * GRP, jnp.maximum(d2[k] + xs[k], 0.0))

    for t in range(nch + 2):
        if t < nch:
            stage1(t)
        if 1 <= t <= nch:
            stage2(t - 1)
        if t >= 2:
            stage3(t - 2)


def kernel(x_nchw, w1, b1, w2, b2):
    N, C, H, W = x_nchw.shape
    HW = H * W

    nb = next(d for d in (16, 8, 4, 2, 1) if N % d == 0)
    grp = max(nb // 2, 1)
    ng = N // nb

    x_packed = x_nchw.reshape(N, C, HW)

    def pack_w(w):
        taps = jnp.transpose(w, (2, 3, 0, 1)).reshape(9, C, C)
        return jnp.transpose(taps, (1, 0, 2)).reshape(C, 9 * C)

    w_all = jnp.stack([pack_w(w1), pack_w(w2)]).astype(jnp.bfloat16)
    b_all = jnp.stack([b1.reshape(C, 1), b2.reshape(C, 1)]).astype(jnp.float32)

    fn = functools.partial(_resblock_kernel, W=W, HW=HW, GRP=grp)

    out = pl.pallas_call(
        fn,
        out_shape=jax.ShapeDtypeStruct((N, C, HW), x_nchw.dtype),
        grid=(ng,),
        in_specs=[
            pl.BlockSpec((nb, C, HW), lambda g: (g, 0, 0)),
            pl.BlockSpec((2, C, 9 * C), lambda g: (0, 0, 0)),
            pl.BlockSpec((2, C, 1), lambda g: (0, 0, 0)),
        ],
        out_specs=pl.BlockSpec((nb, C, HW), lambda g: (g, 0, 0)),
        compiler_params=pltpu.CompilerParams(
            dimension_semantics=("parallel",)),
    )(x_packed, w_all, b_all)

    return out.reshape(N, C, H, W)


# speedup vs baseline: 1.5013x; 1.0004x over previous
import functools

import jax
import jax.numpy as jnp
from jax import lax
from jax.experimental import pallas as pl
from jax.experimental.pallas import tpu as pltpu


def _resblock_kernel(x_ref, w_ref, b_ref, o_ref, *, W, HW, GRP):
    NB, C, _ = x_ref.shape
    L = GRP * HW

    col = lax.broadcasted_iota(jnp.int16, (C, L), 1)
    w_pos = col & jnp.int16(W - 1)
    hw_pos = col & jnp.int16(HW - 1)
    is_w_first = w_pos == 0
    is_w_last = w_pos == W - 1
    is_h_first = hw_pos < W
    is_h_last = hw_pos >= HW - W

    def lane_roll(a, shift):
        return pltpu.roll(a, shift % L, axis=1)

    narrow = HW % 128 == 0 and HW >= 256
    if narrow:
        mh_first = is_h_first[:, :128]
        mh_last = is_h_last[:, HW - 128:HW]

    def h_first_fix(up, down):
        if not narrow:
            return jnp.where(is_h_first, down, up)
        pieces = []
        for i in range(0, L, HW):
            pieces.append(jnp.where(mh_first, down[:, i:i + 128],
                                    up[:, i:i + 128]))
            pieces.append(up[:, i + 128:i + HW])
        return jnp.concatenate(pieces, axis=1)

    def h_last_fix(up, down):
        if not narrow:
            return jnp.where(is_h_last, up, down)
        pieces = []
        for i in range(0, L, HW):
            pieces.append(down[:, i:i + HW - 128])
            pieces.append(jnp.where(mh_last, up[:, i + HW - 128:i + HW],
                                    down[:, i + HW - 128:i + HW]))
        return jnp.concatenate(pieces, axis=1)

    def tap_slab(src):
        left = lane_roll(src, 1)
        right = lane_roll(src, -1)
        w_taps = (jnp.where(is_w_first, right, left), src,
                  jnp.where(is_w_last, left, right))
        cols = []
        for kw in range(3):
            a = w_taps[kw]
            up = lane_roll(a, W)
            down = lane_roll(a, -W)
            cols.append((h_first_fix(up, down), a, h_last_fix(up, down)))
        return jnp.concatenate(
            [cols[kw][kh] for kh in range(3) for kw in range(3)], axis=0)

    def conv3x3(src_bf16, j):
        return jnp.dot(w_ref[j], tap_slab(src_bf16),
                       preferred_element_type=jnp.float32) + b_ref[j]

    nch = NB // GRP
    d1 = [None] * nch
    d2 = [None] * nch

    def stage1(k):
        xb = jnp.concatenate(
            [x_ref[k * GRP + i].astype(jnp.bfloat16) for i in range(GRP)],
            axis=1)
        d1[k] = conv3x3(xb, 0)

    def stage2(k):
        h1 = jnp.maximum(d1[k], 0.0)
        d2[k] = conv3x3(h1.astype(jnp.bfloat16), 1)

    def stage3(k):
        for i in range(GRP):
            o_ref[k * GRP + i] = jnp.maximum(
                d2[k][:, i * HW:(i + 1) * HW] + x_ref[k * GRP + i],
                0.0).astype(o_ref.dtype)

    for t in range(nch + 2):
        if t < nch:
            stage1(t)
        if 1 <= t <= nch:
            stage2(t - 1)
        if t >= 2:
            stage3(t - 2)


def kernel(x_nchw, w1, b1, w2, b2):
    N, C, H, W = x_nchw.shape
    HW = H * W

    nb = next(d for d in (16, 8, 4, 2, 1) if N % d == 0)
    grp = max(nb // 2, 1)
    ng = N // nb

    x_packed = x_nchw.reshape(N, C, HW)

    def pack_w(w):
        taps = jnp.transpose(w, (2, 3, 0, 1)).reshape(9, C, C)
        return jnp.transpose(taps, (1, 0, 2)).reshape(C, 9 * C)

    w_all = jnp.stack([pack_w(w1), pack_w(w2)]).astype(jnp.bfloat16)
    b_all = jnp.stack([b1.reshape(C, 1), b2.reshape(C, 1)]).astype(jnp.float32)

    fn = functools.partial(_resblock_kernel, W=W, HW=HW, GRP=grp)

    out = pl.pallas_call(
        fn,
        out_shape=jax.ShapeDtypeStruct((N, C, HW), x_nchw.dtype),
        grid=(ng,),
        in_specs=[
            pl.BlockSpec((nb, C, HW), lambda g: (g, 0, 0)),
            pl.BlockSpec((2, C, 9 * C), lambda g: (0, 0, 0)),
            pl.BlockSpec((2, C, 1), lambda g: (0, 0, 0)),
        ],
        out_specs=pl.BlockSpec((nb, C, HW), lambda g: (g, 0, 0)),
        compiler_params=pltpu.CompilerParams(
            dimension_semantics=("parallel",)),
    )(x_packed, w_all, b_all)

    return out.reshape(N, C, H, W)


# speedup vs baseline: 1.5406x; 1.0262x over previous
import functools

import jax
import jax.numpy as jnp
from jax import lax
from jax.experimental import pallas as pl
from jax.experimental.pallas import tpu as pltpu


def _resblock_kernel(x_ref, w_ref, b_ref, o_ref, *, W, HW, GRP):
    NB, C, _ = x_ref.shape
    L = GRP * HW

    col = lax.broadcasted_iota(jnp.int16, (C, L), 1)
    w_pos = col & jnp.int16(W - 1)
    hw_pos = col & jnp.int16(HW - 1)
    is_w_first = w_pos == 0
    is_w_last = w_pos == W - 1
    is_h_first = hw_pos < W
    is_h_last = hw_pos >= HW - W

    def lane_roll(a, shift):
        return pltpu.roll(a, shift % L, axis=1)

    narrow = HW % 128 == 0 and HW >= 256
    if narrow:
        mh_first = is_h_first[:, :128]
        mh_last = is_h_last[:, HW - 128:HW]

    def h_first_fix(up, down):
        if not narrow:
            return jnp.where(is_h_first, down, up)
        pieces = []
        for i in range(0, L, HW):
            pieces.append(jnp.where(mh_first, down[:, i:i + 128],
                                    up[:, i:i + 128]))
            pieces.append(up[:, i + 128:i + HW])
        return jnp.concatenate(pieces, axis=1)

    def h_last_fix(up, down):
        if not narrow:
            return jnp.where(is_h_last, up, down)
        pieces = []
        for i in range(0, L, HW):
            pieces.append(down[:, i:i + HW - 128])
            pieces.append(jnp.where(mh_last, up[:, i + HW - 128:i + HW],
                                    down[:, i + HW - 128:i + HW]))
        return jnp.concatenate(pieces, axis=1)

    def tap_slab(src):
        left = lane_roll(src, 1)
        right = lane_roll(src, -1)
        w_taps = (jnp.where(is_w_first, right, left), src,
                  jnp.where(is_w_last, left, right))
        wcat = jnp.concatenate(w_taps, axis=0)
        up_cat = lane_roll(wcat, W)
        down_cat = lane_roll(wcat, -W)
        cols = []
        for kw in range(3):
            a = w_taps[kw]
            up = up_cat[kw * C:(kw + 1) * C]
            down = down_cat[kw * C:(kw + 1) * C]
            cols.append((h_first_fix(up, down), a, h_last_fix(up, down)))
        return jnp.concatenate(
            [cols[kw][kh] for kh in range(3) for kw in range(3)], axis=0)

    def conv3x3(src_bf16, j):
        return jnp.dot(w_ref[j], tap_slab(src_bf16),
                       preferred_element_type=jnp.float32) + b_ref[j]

    nch = NB // GRP
    d1 = [None] * nch
    d2 = [None] * nch

    def stage1(k):
        xb = jnp.concatenate(
            [x_ref[k * GRP + i].astype(jnp.bfloat16) for i in range(GRP)],
            axis=1)
        d1[k] = conv3x3(xb, 0)

    def stage2(k):
        h1 = jnp.maximum(d1[k], 0.0)
        d2[k] = conv3x3(h1.astype(jnp.bfloat16), 1)

    def stage3(k):
        for i in range(GRP):
            o_ref[k * GRP + i] = jnp.maximum(
                d2[k][:, i * HW:(i + 1) * HW] + x_ref[k * GRP + i],
                0.0).astype(o_ref.dtype)

    for t in range(nch + 2):
        if t < nch:
            stage1(t)
        if 1 <= t <= nch:
            stage2(t - 1)
        if t >= 2:
            stage3(t - 2)


def kernel(x_nchw, w1, b1, w2, b2):
    N, C, H, W = x_nchw.shape
    HW = H * W

    nb = next(d for d in (16, 8, 4, 2, 1) if N % d == 0)
    grp = max(nb // 2, 1)
    ng = N // nb

    x_packed = x_nchw.reshape(N, C, HW)

    def pack_w(w):
        taps = jnp.transpose(w, (2, 3, 0, 1)).reshape(9, C, C)
        return jnp.transpose(taps, (1, 0, 2)).reshape(C, 9 * C)

    w_all = jnp.stack([pack_w(w1), pack_w(w2)]).astype(jnp.bfloat16)
    b_all = jnp.stack([b1.reshape(C, 1), b2.reshape(C, 1)]).astype(jnp.float32)

    fn = functools.partial(_resblock_kernel, W=W, HW=HW, GRP=grp)

    out = pl.pallas_call(
        fn,
        out_shape=jax.ShapeDtypeStruct((N, C, HW), x_nchw.dtype),
        grid=(ng,),
        in_specs=[
            pl.BlockSpec((nb, C, HW), lambda g: (g, 0, 0)),
            pl.BlockSpec((2, C, 9 * C), lambda g: (0, 0, 0)),
            pl.BlockSpec((2, C, 1), lambda g: (0, 0, 0)),
        ],
        out_specs=pl.BlockSpec((nb, C, HW), lambda g: (g, 0, 0)),
        compiler_params=pltpu.CompilerParams(
            dimension_semantics=("parallel",)),
    )(x_packed, w_all, b_all)

    return out.reshape(N, C, H, W)


# speedup vs baseline: 1.5424x; 1.0011x over previous
import functools

import jax
import jax.numpy as jnp
from jax import lax
from jax.experimental import pallas as pl
from jax.experimental.pallas import tpu as pltpu


def _resblock_kernel(x_ref, w_ref, b_ref, o_ref, *, W, HW, GRP):
    NB, C, _ = x_ref.shape
    L = GRP * HW

    col = lax.broadcasted_iota(jnp.int16, (C, L), 1)
    w_pos = col & jnp.int16(W - 1)
    is_w_first = w_pos == 0
    is_w_last = w_pos == W - 1

    def lane_roll(a, shift):
        return pltpu.roll(a, shift % L, axis=1)

    narrow = HW % 128 == 0 and HW >= 256
    if narrow:
        lane = lax.broadcasted_iota(jnp.int16, (3 * C, 128), 1)
        mh_first = lane < W
        mh_last = lane >= 128 - W
    else:
        col3 = lax.broadcasted_iota(jnp.int16, (3 * C, L), 1)
        hw3 = col3 & jnp.int16(HW - 1)
        ih3_first = hw3 < W
        ih3_last = hw3 >= HW - W

    def h_first_fix(up, down):
        if not narrow:
            return jnp.where(ih3_first, down, up)
        pieces = []
        for i in range(0, L, HW):
            pieces.append(jnp.where(mh_first, down[:, i:i + 128],
                                    up[:, i:i + 128]))
            pieces.append(up[:, i + 128:i + HW])
        return jnp.concatenate(pieces, axis=1)

    def h_last_fix(up, down):
        if not narrow:
            return jnp.where(ih3_last, up, down)
        pieces = []
        for i in range(0, L, HW):
            pieces.append(down[:, i:i + HW - 128])
            pieces.append(jnp.where(mh_last, up[:, i + HW - 128:i + HW],
                                    down[:, i + HW - 128:i + HW]))
        return jnp.concatenate(pieces, axis=1)

    def tap_slab(src):
        left = lane_roll(src, 1)
        right = lane_roll(src, -1)
        wcat = jnp.concatenate(
            (jnp.where(is_w_first, right, left), src,
             jnp.where(is_w_last, left, right)), axis=0)
        up_cat = lane_roll(wcat, W)
        down_cat = lane_roll(wcat, -W)
        return jnp.concatenate(
            (h_first_fix(up_cat, down_cat), wcat,
             h_last_fix(up_cat, down_cat)), axis=0)

    def conv3x3(src_bf16, j):
        return jnp.dot(w_ref[j], tap_slab(src_bf16),
                       preferred_element_type=jnp.float32) + b_ref[j]

    nch = NB // GRP
    d1 = [None] * nch
    d2 = [None] * nch

    def stage1(k):
        xb = jnp.concatenate(
            [x_ref[k * GRP + i].astype(jnp.bfloat16) for i in range(GRP)],
            axis=1)
        d1[k] = conv3x3(xb, 0)

    def stage2(k):
        h1 = jnp.maximum(d1[k], 0.0)
        d2[k] = conv3x3(h1.astype(jnp.bfloat16), 1)

    def stage3(k):
        for i in range(GRP):
            o_ref[k * GRP + i] = jnp.maximum(
                d2[k][:, i * HW:(i + 1) * HW] + x_ref[k * GRP + i],
                0.0).astype(o_ref.dtype)

    for t in range(nch + 2):
        if t < nch:
            stage1(t)
        if 1 <= t <= nch:
            stage2(t - 1)
        if t >= 2:
            stage3(t - 2)


def kernel(x_nchw, w1, b1, w2, b2):
    N, C, H, W = x_nchw.shape
    HW = H * W

    nb = next(d for d in (16, 8, 4, 2, 1) if N % d == 0)
    grp = max(nb // 2, 1)
    ng = N // nb

    x_packed = x_nchw.reshape(N, C, HW)

    def pack_w(w):
        taps = jnp.transpose(w, (2, 3, 0, 1)).reshape(9, C, C)
        return jnp.transpose(taps, (1, 0, 2)).reshape(C, 9 * C)

    w_all = jnp.stack([pack_w(w1), pack_w(w2)]).astype(jnp.bfloat16)
    b_all = jnp.stack([b1.reshape(C, 1), b2.reshape(C, 1)]).astype(jnp.float32)

    fn = functools.partial(_resblock_kernel, W=W, HW=HW, GRP=grp)

    out = pl.pallas_call(
        fn,
        out_shape=jax.ShapeDtypeStruct((N, C, HW), x_nchw.dtype),
        grid=(ng,),
        in_specs=[
            pl.BlockSpec((nb, C, HW), lambda g: (g, 0, 0)),
            pl.BlockSpec((2, C, 9 * C), lambda g: (0, 0, 0)),
            pl.BlockSpec((2, C, 1), lambda g: (0, 0, 0)),
        ],
        out_specs=pl.BlockSpec((nb, C, HW), lambda g: (g, 0, 0)),
        compiler_params=pltpu.CompilerParams(
            dimension_semantics=("parallel",)),
    )(x_packed, w_all, b_all)

    return out.reshape(N, C, H, W)
